# baseline (device time: 16579 ns/iter reference)
import jax
import jax.numpy as jnp
from jax import lax
from jax.experimental import pallas as pl
from jax.experimental.pallas import tpu as pltpu

N_DEV = 8

MASKS = (6, 2, 5, 7, 1, 3, 4)


def kernel(x):
    _, m, n = x.shape
    seg = m // N_DEV
    nsb = 2
    sb_rows = seg // nsb

    def body(
        x_ref,
        out_ref,
        x16,
        seg16,
        acc32,
        rs_buf,
        ag_buf,
        rs_send,
        rs_recv,
        ag_send,
        ag_recv,
        out_sems,
    ):
        me = lax.axis_index("i")

        x16[:, :] = x_ref[0, :, :].astype(jnp.bfloat16)

        barrier_sem = pltpu.get_barrier_semaphore()
        for mk in MASKS:
            pl.semaphore_signal(
                barrier_sem,
                inc=1,
                device_id=(me ^ mk,),
                device_id_type=pl.DeviceIdType.MESH,
            )
        pl.semaphore_wait(barrier_sem, N_DEV - 1)

        rs_descs = {}
        for sb in range(nsb):
            for mk in MASKS:
                q = me ^ mk
                rdma = pltpu.make_async_remote_copy(
                    src_ref=x16.at[pl.ds(q * seg + sb * sb_rows, sb_rows), :],
                    dst_ref=rs_buf.at[sb, mk - 1],
                    send_sem=rs_send.at[sb * (N_DEV - 1) + mk - 1],
                    recv_sem=rs_recv.at[sb * (N_DEV - 1) + mk - 1],
                    device_id=(q,),
                    device_id_type=pl.DeviceIdType.MESH,
                )
                rdma.start()
                rs_descs[sb, mk] = rdma

        out_copies = []
        ag_descs = {}
        for sb in range(nsb):
            rows = pl.ds(me * seg + sb * sb_rows, sb_rows)
            acc = x_ref[0, rows, :]
            for mk in MASKS:
                rs_descs[sb, mk].wait_recv()
                acc = acc + rs_buf[sb, mk - 1, :, :].astype(jnp.float32)
            acc32[rows, :] = acc
            seg16[pl.ds(sb * sb_rows, sb_rows), :] = acc.astype(jnp.bfloat16)
            cp = pltpu.make_async_copy(
                acc32.at[rows, :], out_ref.at[rows, :], out_sems.at[sb * N_DEV + 7]
            )
            cp.start()
            out_copies.append(cp)
            for mk in MASKS:
                q = me ^ mk
                rdma = pltpu.make_async_remote_copy(
                    src_ref=seg16.at[pl.ds(sb * sb_rows, sb_rows), :],
                    dst_ref=ag_buf.at[sb, mk - 1],
                    send_sem=ag_send.at[sb * (N_DEV - 1) + mk - 1],
                    recv_sem=ag_recv.at[sb * (N_DEV - 1) + mk - 1],
                    device_id=(q,),
                    device_id_type=pl.DeviceIdType.MESH,
                )
                rdma.start()
                ag_descs[sb, mk] = rdma

        for sb in range(nsb):
            for mk in MASKS:
                q = me ^ mk
                rows = pl.ds(q * seg + sb * sb_rows, sb_rows)
                ag_descs[sb, mk].wait_recv()
                acc32[rows, :] = ag_buf[sb, mk - 1, :, :].astype(jnp.float32)
                cp = pltpu.make_async_copy(
                    acc32.at[rows, :],
                    out_ref.at[rows, :],
                    out_sems.at[sb * N_DEV + mk - 1],
                )
                cp.start()
                out_copies.append(cp)

        for rdma in rs_descs.values():
            rdma.wait_send()
        for rdma in ag_descs.values():
            rdma.wait_send()
        for cp in out_copies:
            cp.wait()

    return pl.pallas_call(
        body,
        out_shape=jax.ShapeDtypeStruct((m, n), x.dtype),
        in_specs=[pl.BlockSpec(memory_space=pltpu.VMEM)],
        out_specs=pl.BlockSpec(memory_space=pl.ANY),
        scratch_shapes=[
            pltpu.VMEM((m, n), jnp.bfloat16),
            pltpu.VMEM((seg, n), jnp.bfloat16),
            pltpu.VMEM((m, n), jnp.float32),
            pltpu.VMEM((nsb, N_DEV - 1, sb_rows, n), jnp.bfloat16),
            pltpu.VMEM((nsb, N_DEV - 1, sb_rows, n), jnp.bfloat16),
            pltpu.SemaphoreType.DMA((nsb * (N_DEV - 1),)),
            pltpu.SemaphoreType.DMA((nsb * (N_DEV - 1),)),
            pltpu.SemaphoreType.DMA((nsb * (N_DEV - 1),)),
            pltpu.SemaphoreType.DMA((nsb * (N_DEV - 1),)),
            pltpu.SemaphoreType.DMA((nsb * N_DEV,)),
        ],
        compiler_params=pltpu.CompilerParams(collective_id=0),
    )(x)


# device time: 15869 ns/iter; 1.0447x vs baseline; 1.0447x over previous
import jax
import jax.numpy as jnp
from jax import lax
from jax.experimental import pallas as pl
from jax.experimental.pallas import tpu as pltpu

N_DEV = 8

MASKS = (6, 2, 5, 7, 1, 3, 4)


def kernel(x):
    _, m, n = x.shape
    seg = m // N_DEV
    nsb = 2
    sb_rows = seg // nsb

    def body(
        x_ref,
        out_ref,
        x16,
        seg16,
        rs_buf,
        ag_buf,
        rs_send,
        rs_recv,
        ag_send,
        ag_recv,
    ):
        me = lax.axis_index("i")

        x16[:, :] = x_ref[0, :, :].astype(jnp.bfloat16)

        barrier_sem = pltpu.get_barrier_semaphore()
        for mk in MASKS:
            pl.semaphore_signal(
                barrier_sem,
                inc=1,
                device_id=(me ^ mk,),
                device_id_type=pl.DeviceIdType.MESH,
            )
        pl.semaphore_wait(barrier_sem, N_DEV - 1)

        rs_descs = {}
        for sb in range(nsb):
            for mk in MASKS:
                q = me ^ mk
                rdma = pltpu.make_async_remote_copy(
                    src_ref=x16.at[pl.ds(q * seg + sb * sb_rows, sb_rows), :],
                    dst_ref=rs_buf.at[sb, mk - 1],
                    send_sem=rs_send.at[sb * (N_DEV - 1) + mk - 1],
                    recv_sem=rs_recv.at[sb * (N_DEV - 1) + mk - 1],
                    device_id=(q,),
                    device_id_type=pl.DeviceIdType.MESH,
                )
                rdma.start()
                rs_descs[sb, mk] = rdma

        ag_descs = {}
        for sb in range(nsb):
            acc = x_ref[0, pl.ds(me * seg + sb * sb_rows, sb_rows), :]
            for mk in MASKS:
                rs_descs[sb, mk].wait_recv()
                acc = acc + rs_buf[sb, mk - 1, :, :].astype(jnp.float32)
            out_ref[pl.ds(me * seg + sb * sb_rows, sb_rows), :] = acc
            seg16[pl.ds(sb * sb_rows, sb_rows), :] = acc.astype(jnp.bfloat16)
            for mk in MASKS:
                q = me ^ mk
                rdma = pltpu.make_async_remote_copy(
                    src_ref=seg16.at[pl.ds(sb * sb_rows, sb_rows), :],
                    dst_ref=ag_buf.at[sb, mk - 1],
                    send_sem=ag_send.at[sb * (N_DEV - 1) + mk - 1],
                    recv_sem=ag_recv.at[sb * (N_DEV - 1) + mk - 1],
                    device_id=(q,),
                    device_id_type=pl.DeviceIdType.MESH,
                )
                rdma.start()
                ag_descs[sb, mk] = rdma

        for sb in range(nsb):
            for mk in MASKS:
                q = me ^ mk
                ag_descs[sb, mk].wait_recv()
                out_ref[pl.ds(q * seg + sb * sb_rows, sb_rows), :] = ag_buf[
                    sb, mk - 1, :, :
                ].astype(jnp.float32)

        for rdma in rs_descs.values():
            rdma.wait_send()
        for rdma in ag_descs.values():
            rdma.wait_send()

    return pl.pallas_call(
        body,
        out_shape=jax.ShapeDtypeStruct((m, n), x.dtype),
        in_specs=[pl.BlockSpec(memory_space=pltpu.VMEM)],
        out_specs=pl.BlockSpec(memory_space=pltpu.VMEM),
        scratch_shapes=[
            pltpu.VMEM((m, n), jnp.bfloat16),
            pltpu.VMEM((seg, n), jnp.bfloat16),
            pltpu.VMEM((nsb, N_DEV - 1, sb_rows, n), jnp.bfloat16),
            pltpu.VMEM((nsb, N_DEV - 1, sb_rows, n), jnp.bfloat16),
            pltpu.SemaphoreType.DMA((nsb * (N_DEV - 1),)),
            pltpu.SemaphoreType.DMA((nsb * (N_DEV - 1),)),
            pltpu.SemaphoreType.DMA((nsb * (N_DEV - 1),)),
            pltpu.SemaphoreType.DMA((nsb * (N_DEV - 1),)),
        ],
        compiler_params=pltpu.CompilerParams(collective_id=0),
    )(x)
